# Initial kernel scaffold; baseline (speedup 1.0000x reference)
import jax
import jax.numpy as jnp
from jax.experimental import pallas as pl


def kernel(obs_pos, obs_mask, embedding_table):
    raise NotImplementedError("write your pallas kernel here")


# speedup vs baseline: 1.7506x; 1.7506x over previous
import functools

import jax
import jax.numpy as jnp
from jax import lax
from jax.experimental import pallas as pl
from jax.experimental.pallas import tpu as pltpu
from jax.experimental.pallas import tpu_sc as plsc

_W = 768
_LANES = 16
_VPR = _W // _LANES


@functools.lru_cache(maxsize=None)
def _build(N, V):
    NC, NS = 2, 16
    NW = NC * NS
    RPW = N // NW
    C = 64
    NCH = RPW // C

    mesh = plsc.VectorSubcoreMesh(core_axis_name="c", subcore_axis_name="s")

    @functools.partial(
        pl.kernel,
        out_type=jax.ShapeDtypeStruct((N, _W), jnp.float32),
        mesh=mesh,
        scratch_types=[
            pltpu.VMEM((NCH, C), jnp.int32),
            pltpu.VMEM((RPW,), jnp.int32),
            pltpu.VMEM((C, _W), jnp.float32),
            pltpu.SemaphoreType.DMA,
        ],
    )
    def k(table_hbm, idx_hbm, mask_hbm, out_hbm, idx_v, mask_v, buf, sem):
        wid = lax.axis_index("s") * NC + lax.axis_index("c")
        base = wid * RPW
        pltpu.sync_copy(idx_hbm.at[wid], idx_v)
        pltpu.sync_copy(mask_hbm.at[wid], mask_v)
        zeros = jnp.zeros((_LANES,), jnp.float32)
        def chunk_body(g, carry):
            pltpu.async_copy(table_hbm.at[idx_v.at[g]], buf, sem).wait()

            def q_body(q, carry2):
                mvec = mask_v[pl.ds(g * C + q * _LANES, _LANES)]
                for i in range(_LANES):
                    r = q * _LANES + i

                    @pl.when(mvec[i] == 0)
                    def _(r=r):
                        for j in range(_VPR):
                            buf[r, pl.ds(j * _LANES, _LANES)] = zeros

                return carry2

            lax.fori_loop(0, C // _LANES, q_body, 0)
            pltpu.sync_copy(buf, out_hbm.at[pl.ds(base + g * C, C)])
            return carry

        lax.fori_loop(0, NCH, chunk_body, 0)

    return k


def kernel(obs_pos, obs_mask, embedding_table):
    B, L = obs_pos.shape
    V, W = embedding_table.shape
    N = B * L
    NW = 32
    RPW = N // NW
    C = 64
    idx = obs_pos.astype(jnp.int32).reshape(NW, RPW // C, C)
    mask = obs_mask.astype(jnp.int32).reshape(NW, RPW)
    out = _build(N, V)(embedding_table, idx, mask)
    return out.reshape(B, L, W)


# speedup vs baseline: 2.1728x; 1.2412x over previous
import functools

import jax
import jax.numpy as jnp
from jax import lax
from jax.experimental import pallas as pl
from jax.experimental.pallas import tpu as pltpu
from jax.experimental.pallas import tpu_sc as plsc

_W = 768
_LANES = 16
_VPR = _W // _LANES
_NBUF = 4


@functools.lru_cache(maxsize=None)
def _build(N, V):
    NC, NS = 2, 16
    NW = NC * NS
    RPW = N // NW
    C = 32
    NCH = RPW // C

    mesh = plsc.VectorSubcoreMesh(core_axis_name="c", subcore_axis_name="s")

    @functools.partial(
        pl.kernel,
        out_type=jax.ShapeDtypeStruct((N, _W), jnp.float32),
        mesh=mesh,
        scratch_types=[
            pltpu.VMEM((NCH, C), jnp.int32),
            pltpu.VMEM((RPW,), jnp.int32),
        ]
        + [pltpu.VMEM((C, _W), jnp.float32) for _ in range(_NBUF)]
        + [pltpu.SemaphoreType.DMA for _ in range(2 * _NBUF)],
    )
    def k(table_hbm, idx_hbm, mask_hbm, out_hbm, idx_v, mask_v, *bs):
        bufs = bs[:_NBUF]
        gsem = bs[_NBUF:2 * _NBUF]
        ssem = bs[2 * _NBUF:]
        wid = lax.axis_index("s") * NC + lax.axis_index("c")
        base = wid * RPW
        pltpu.sync_copy(idx_hbm.at[wid], idx_v)
        pltpu.sync_copy(mask_hbm.at[wid], mask_v)
        zeros = jnp.zeros((_LANES,), jnp.float32)

        def gstart(g, b):
            pltpu.async_copy(table_hbm.at[idx_v.at[g]], bufs[b], gsem[b])

        def gwait(g, b):
            pltpu.make_async_copy(
                table_hbm.at[idx_v.at[g]], bufs[b], gsem[b]).wait()

        def sstart(g, b):
            pltpu.async_copy(
                bufs[b], out_hbm.at[pl.ds(base + g * C, C)], ssem[b])

        def swait(g, b):
            pltpu.make_async_copy(
                bufs[b], out_hbm.at[pl.ds(base + g * C, C)], ssem[b]).wait()

        gstart(0, 0)
        gstart(1, 1)

        def outer(o, carry):
            for b in range(_NBUF):
                g = o * _NBUF + b
                gwait(g, b)

                def q_body(q, carry2, b=b, g=g):
                    buf = bufs[b]
                    mvec = mask_v[pl.ds(g * C + q * _LANES, _LANES)]
                    for i in range(_LANES):
                        r = q * _LANES + i

                        @pl.when(mvec[i] == 0)
                        def _(r=r, buf=buf):
                            for j in range(_VPR):
                                buf[r, pl.ds(j * _LANES, _LANES)] = zeros

                    return carry2

                lax.fori_loop(0, C // _LANES, q_body, 0)
                sstart(g, b)
                b2 = (b + 2) % _NBUF

                @pl.when(g >= 2)
                def _(g=g, b2=b2):
                    swait(g - 2, b2)

                @pl.when(g + 2 < NCH)
                def _(g=g, b2=b2):
                    gstart(g + 2, b2)

            return carry

        lax.fori_loop(0, NCH // _NBUF, outer, 0)
        swait(NCH - 2, (NCH - 2) % _NBUF)
        swait(NCH - 1, (NCH - 1) % _NBUF)

    return k


def kernel(obs_pos, obs_mask, embedding_table):
    B, L = obs_pos.shape
    V, W = embedding_table.shape
    N = B * L
    NW = 32
    RPW = N // NW
    C = 32
    idx = obs_pos.astype(jnp.int32).reshape(NW, RPW // C, C)
    mask = obs_mask.astype(jnp.int32).reshape(NW, RPW)
    out = _build(N, V)(embedding_table, idx, mask)
    return out.reshape(B, L, W)
